# baseline (speedup 1.0000x reference)
import jax
import jax.numpy as jnp
from jax.experimental import pallas as pl
from jax.experimental.pallas import tpu as pltpu

_BATCH = 1
_SEQ = 8192
_HEADS = 32
_HEAD_DIM = 128
_Q = 16
_ROW = _HEADS * _HEAD_DIM

_CHR = 256
_M = 8
_L = 4
_NC = _SEQ // _CHR
_T = 2 * _NC


def _body(pos_ref, kc, vc, kv_ref, vv_ref, ko, vo, buf, ldsem, stsem):
    def parts(c):
        if c < _NC:
            return kc, ko, kv_ref, c
        return vc, vo, vv_ref, c - _NC

    def load(c):
        src, _, _, i = parts(c)
        s = c % _M
        return pltpu.make_async_copy(
            src.at[pl.ds(i * _CHR, _CHR)], buf.at[s], ldsem.at[s])

    def store(c):
        _, dst, _, i = parts(c)
        s = c % _M
        return pltpu.make_async_copy(
            buf.at[s], dst.at[pl.ds(i * _CHR, _CHR)], stsem.at[s])

    def scatter(c):
        _, _, val, i = parts(c)
        s = c % _M
        base = i * _CHR
        for j in range(_Q):
            p = pos_ref[j]

            @pl.when(jnp.logical_and(p >= base, p < base + _CHR))
            def _():
                buf[s, pl.ds(p - base, 1), :] = val[pl.ds(j, 1), :]

    waited = set()
    for c in range(min(_L, _T)):
        load(c).start()
    for c in range(_T):
        pre = c + _L
        if pre < _T:
            if pre - _M >= 0:
                store(pre - _M).wait()
                waited.add(pre - _M)
            load(pre).start()
        load(c).wait()
        scatter(c)
        store(c).start()
    for c in range(_T):
        if c not in waited:
            store(c).wait()


def kernel(k_cache, v_cache, input_pos, k_val, v_val):
    kc = k_cache.reshape(_SEQ, _ROW)
    vc = v_cache.reshape(_SEQ, _ROW)
    kv = k_val.reshape(_Q, _ROW)
    vv = v_val.reshape(_Q, _ROW)
    pos = input_pos.astype(jnp.int32)

    out_k, out_v = pl.pallas_call(
        _body,
        in_specs=[
            pl.BlockSpec(memory_space=pltpu.SMEM),
            pl.BlockSpec(memory_space=pl.MemorySpace.ANY),
            pl.BlockSpec(memory_space=pl.MemorySpace.ANY),
            pl.BlockSpec(memory_space=pltpu.VMEM),
            pl.BlockSpec(memory_space=pltpu.VMEM),
        ],
        out_specs=[
            pl.BlockSpec(memory_space=pl.MemorySpace.ANY),
            pl.BlockSpec(memory_space=pl.MemorySpace.ANY),
        ],
        out_shape=[
            jax.ShapeDtypeStruct((_SEQ, _ROW), jnp.float32),
            jax.ShapeDtypeStruct((_SEQ, _ROW), jnp.float32),
        ],
        scratch_shapes=[
            pltpu.VMEM((_M, _CHR, _ROW), jnp.float32),
            pltpu.SemaphoreType.DMA((_M,)),
            pltpu.SemaphoreType.DMA((_M,)),
        ],
    )(pos, kc, vc, kv, vv)

    return (
        out_k.reshape(_BATCH, _SEQ, _HEADS, _HEAD_DIM),
        out_v.reshape(_BATCH, _SEQ, _HEADS, _HEAD_DIM),
    )


# speedup vs baseline: 3.9761x; 3.9761x over previous
import jax
import jax.numpy as jnp
from jax.experimental import pallas as pl
from jax.experimental.pallas import tpu as pltpu

_BATCH = 1
_SEQ = 8192
_HEADS = 32
_HEAD_DIM = 128
_Q = 16

_CHR = 256
_M = 8
_L = 4
_NC = _SEQ // _CHR
_T = 2 * _NC


def _body(pos_ref, kc, vc, kv_ref, vv_ref, ko, vo, buf, ldsem, stsem):
    def parts(c):
        if c < _NC:
            return kc, ko, kv_ref, c
        return vc, vo, vv_ref, c - _NC

    def load(c):
        src, _, _, i = parts(c)
        s = c % _M
        return pltpu.make_async_copy(
            src.at[0, pl.ds(i * _CHR, _CHR)], buf.at[s], ldsem.at[s])

    def store(c):
        _, dst, _, i = parts(c)
        s = c % _M
        return pltpu.make_async_copy(
            buf.at[s], dst.at[0, pl.ds(i * _CHR, _CHR)], stsem.at[s])

    def scatter(c):
        _, _, val, i = parts(c)
        s = c % _M
        base = i * _CHR
        for j in range(_Q):
            p = pos_ref[j]

            @pl.when(jnp.logical_and(p >= base, p < base + _CHR))
            def _():
                buf[s, pl.ds(p - base, 1)] = val[0, pl.ds(j, 1)]

    waited = set()
    for c in range(min(_L, _T)):
        load(c).start()
    for c in range(_T):
        pre = c + _L
        if pre < _T:
            if pre - _M >= 0:
                store(pre - _M).wait()
                waited.add(pre - _M)
            load(pre).start()
        load(c).wait()
        scatter(c)
        store(c).start()
    for c in range(_T):
        if c not in waited:
            store(c).wait()


def kernel(k_cache, v_cache, input_pos, k_val, v_val):
    pos = input_pos.astype(jnp.int32)

    out_k, out_v = pl.pallas_call(
        _body,
        in_specs=[
            pl.BlockSpec(memory_space=pltpu.SMEM),
            pl.BlockSpec(memory_space=pl.MemorySpace.ANY),
            pl.BlockSpec(memory_space=pl.MemorySpace.ANY),
            pl.BlockSpec(memory_space=pltpu.VMEM),
            pl.BlockSpec(memory_space=pltpu.VMEM),
        ],
        out_specs=[
            pl.BlockSpec(memory_space=pl.MemorySpace.ANY),
            pl.BlockSpec(memory_space=pl.MemorySpace.ANY),
        ],
        out_shape=[
            jax.ShapeDtypeStruct((_BATCH, _SEQ, _HEADS, _HEAD_DIM), jnp.float32),
            jax.ShapeDtypeStruct((_BATCH, _SEQ, _HEADS, _HEAD_DIM), jnp.float32),
        ],
        scratch_shapes=[
            pltpu.VMEM((_M, _CHR, _HEADS, _HEAD_DIM), jnp.float32),
            pltpu.SemaphoreType.DMA((_M,)),
            pltpu.SemaphoreType.DMA((_M,)),
        ],
    )(pos, k_cache, v_cache, k_val, v_val)

    return (out_k, out_v)


# speedup vs baseline: 3.9800x; 1.0010x over previous
import jax
import jax.numpy as jnp
from jax.experimental import pallas as pl
from jax.experimental.pallas import tpu as pltpu

_BATCH = 1
_SEQ = 8192
_HEADS = 32
_HEAD_DIM = 128
_Q = 16

_CHR = 512
_M = 6
_L = 3
_NC = _SEQ // _CHR
_T = 2 * _NC


def _body(pos_ref, kc, vc, kv_ref, vv_ref, ko, vo, buf, ldsem, stsem):
    def parts(c):
        if c < _NC:
            return kc, ko, kv_ref, c
        return vc, vo, vv_ref, c - _NC

    def load(c):
        src, _, _, i = parts(c)
        s = c % _M
        return pltpu.make_async_copy(
            src.at[0, pl.ds(i * _CHR, _CHR)], buf.at[s], ldsem.at[s])

    def store(c):
        _, dst, _, i = parts(c)
        s = c % _M
        return pltpu.make_async_copy(
            buf.at[s], dst.at[0, pl.ds(i * _CHR, _CHR)], stsem.at[s])

    def scatter(c):
        _, _, val, i = parts(c)
        s = c % _M
        base = i * _CHR
        for j in range(_Q):
            p = pos_ref[j]

            @pl.when(jnp.logical_and(p >= base, p < base + _CHR))
            def _():
                buf[s, pl.ds(p - base, 1)] = val[0, pl.ds(j, 1)]

    waited = set()
    for c in range(min(_L, _T)):
        load(c).start()
    for c in range(_T):
        pre = c + _L
        if pre < _T:
            if pre - _M >= 0:
                store(pre - _M).wait()
                waited.add(pre - _M)
            load(pre).start()
        load(c).wait()
        scatter(c)
        store(c).start()
    for c in range(_T):
        if c not in waited:
            store(c).wait()


def kernel(k_cache, v_cache, input_pos, k_val, v_val):
    pos = input_pos.astype(jnp.int32)

    out_k, out_v = pl.pallas_call(
        _body,
        in_specs=[
            pl.BlockSpec(memory_space=pltpu.SMEM),
            pl.BlockSpec(memory_space=pl.MemorySpace.ANY),
            pl.BlockSpec(memory_space=pl.MemorySpace.ANY),
            pl.BlockSpec(memory_space=pltpu.VMEM),
            pl.BlockSpec(memory_space=pltpu.VMEM),
        ],
        out_specs=[
            pl.BlockSpec(memory_space=pl.MemorySpace.ANY),
            pl.BlockSpec(memory_space=pl.MemorySpace.ANY),
        ],
        out_shape=[
            jax.ShapeDtypeStruct((_BATCH, _SEQ, _HEADS, _HEAD_DIM), jnp.float32),
            jax.ShapeDtypeStruct((_BATCH, _SEQ, _HEADS, _HEAD_DIM), jnp.float32),
        ],
        scratch_shapes=[
            pltpu.VMEM((_M, _CHR, _HEADS, _HEAD_DIM), jnp.float32),
            pltpu.SemaphoreType.DMA((_M,)),
            pltpu.SemaphoreType.DMA((_M,)),
        ],
    )(pos, k_cache, v_cache, k_val, v_val)

    return (out_k, out_v)
